# baseline (speedup 1.0000x reference)
import functools

import jax
import jax.numpy as jnp
from jax import lax
from jax.experimental import pallas as pl
from jax.experimental.pallas import tpu as pltpu
from jax.experimental.pallas import tpu_sc as plsc

_B, _C, _N, _K = 2, 128, 10000, 32
_P = _B * _N
_LANES = 16
_NW = 32
_EPC = 128
_PPC = _EPC // _K
_CPW = 160
_P_PAD = _NW * _CPW * _PPC
_FLUSH = 16
_ROWBLK = 1000


def _attn_weights_body(x_ref, w_ref, g_ref):
    x = x_ref[...]
    h = lax.dot_general(x, w_ref[...], (((1,), (1,)), ((), ())),
                        preferred_element_type=jnp.float32)
    h = h - jnp.max(h, axis=1, keepdims=True)
    e = jnp.exp(h)
    sm = e / jnp.sum(e, axis=1, keepdims=True)
    g_ref[...] = sm * x


def _attn_weights(feat, w_mlp):
    return pl.pallas_call(
        _attn_weights_body,
        grid=(_P // _ROWBLK,),
        in_specs=[
            pl.BlockSpec((_ROWBLK, _C), lambda i: (i, 0)),
            pl.BlockSpec((_C, _C), lambda i: (0, 0)),
        ],
        out_specs=pl.BlockSpec((_ROWBLK, _C), lambda i: (i, 0)),
        out_shape=jax.ShapeDtypeStruct((_P, _C), jnp.float32),
    )(feat, w_mlp)


_NBUF = 4
_CPW0 = 256
_CPW1 = 2 * _CPW - _CPW0


def _segsum_body(g_hbm, idx_hbm, out_hbm, idx_v, bufs, out_v, sems):
    cid = lax.axis_index("c")
    sid = lax.axis_index("s")
    chunk0 = sid * (2 * _CPW) + cid * _CPW0
    my_cpw = jnp.where(cid == 0, _CPW0, _CPW1)

    idx_row0 = pl.multiple_of(chunk0, 8)

    @pl.when(cid == 0)
    def _():
        pltpu.sync_copy(idx_hbm.at[pl.ds(idx_row0, _CPW0)], idx_v)

    @pl.when(cid != 0)
    def _():
        pltpu.sync_copy(idx_hbm.at[pl.ds(idx_row0, _CPW1)],
                        idx_v.at[pl.ds(0, _CPW1)])

    def start_gather(c, b):
        pltpu.async_copy(g_hbm.at[idx_v.at[c]], bufs[b], sems[b])

    def wait_gather(b):
        pltpu.make_async_copy(g_hbm.at[idx_v.at[0]], bufs[b], sems[b]).wait()

    def reduce_chunk(c, buf):
        orow = (c % _FLUSH) * _PPC
        for p in range(_PPC):
            accs = [buf[p * _K, pl.ds(l * _LANES, _LANES)]
                    for l in range(_C // _LANES)]
            for j in range(1, _K):
                accs = [accs[l] + buf[p * _K + j, pl.ds(l * _LANES, _LANES)]
                        for l in range(_C // _LANES)]
            for l in range(_C // _LANES):
                out_v[orow + p, pl.ds(l * _LANES, _LANES)] = accs[l]

    for b in range(_NBUF):
        start_gather(b, b)

    @pl.loop(0, _CPW0, step=_NBUF)
    def _group(c):
        for b in range(_NBUF):
            ch = c + b

            @pl.when(ch < my_cpw)
            def _():
                wait_gather(b)
                reduce_chunk(ch, bufs[b])

                @pl.when(ch + _NBUF < my_cpw)
                def _():
                    start_gather(ch + _NBUF, b)

            @pl.when((ch < my_cpw) & (ch % _FLUSH == _FLUSH - 1))
            def _():
                first_pt = pl.multiple_of(
                    (chunk0 + ch - (_FLUSH - 1)) * _PPC, 8)
                pltpu.sync_copy(out_v,
                                out_hbm.at[pl.ds(first_pt, _FLUSH * _PPC)])


def _segsum(g, idx_rows):
    mesh = plsc.VectorSubcoreMesh(core_axis_name="c", subcore_axis_name="s")

    def body(g_hbm, idx_hbm, out_hbm, idx_v, b0, b1, b2, b3, out_v,
             s0, s1, s2, s3):
        _segsum_body(g_hbm, idx_hbm, out_hbm, idx_v, (b0, b1, b2, b3),
                     out_v, (s0, s1, s2, s3))

    kern = pl.kernel(
        body,
        out_type=jax.ShapeDtypeStruct((_P_PAD, _C), jnp.float32),
        mesh=mesh,
        scratch_types=[
            pltpu.VMEM((_CPW0, _EPC), jnp.int32),
            pltpu.VMEM((_EPC, _C), jnp.float32),
            pltpu.VMEM((_EPC, _C), jnp.float32),
            pltpu.VMEM((_EPC, _C), jnp.float32),
            pltpu.VMEM((_EPC, _C), jnp.float32),
            pltpu.VMEM((_FLUSH * _PPC, _C), jnp.float32),
            pltpu.SemaphoreType.DMA,
            pltpu.SemaphoreType.DMA,
            pltpu.SemaphoreType.DMA,
            pltpu.SemaphoreType.DMA,
        ],
    )
    return kern(g, idx_rows)


def _proj_stats_body(a_ref, wc_ref, bc_ref, s_ref, q_ref):
    i = pl.program_id(0)
    a = a_ref[...].astype(jnp.float32)
    x = lax.dot_general(a, wc_ref[...], (((1,), (1,)), ((), ())),
                        preferred_element_type=jnp.float32) + bc_ref[...]

    @pl.when(i == 0)
    def _():
        s_ref[...] = jnp.zeros_like(s_ref)
        q_ref[...] = jnp.zeros_like(q_ref)

    s_ref[...] += jnp.sum(x, axis=0, keepdims=True)
    q_ref[...] += jnp.sum(x * x, axis=0, keepdims=True)


def _proj_norm_body(a_ref, wc_ref, bc_ref, s_ref, q_ref, gm_ref, bt_ref,
                    o_ref):
    a = a_ref[...].astype(jnp.float32)
    x = lax.dot_general(a, wc_ref[...], (((1,), (1,)), ((), ())),
                        preferred_element_type=jnp.float32) + bc_ref[...]
    inv_n = 1.0 / _P
    mean = s_ref[...] * inv_n
    var = q_ref[...] * inv_n - mean * mean
    scale = gm_ref[...] * lax.rsqrt(var + 1e-5)
    o_ref[...] = (x - mean) * scale + bt_ref[...]


def _proj_bn(agg, w_conv, b_conv, gamma, beta):
    grid = (_P // _ROWBLK,)
    row_spec = pl.BlockSpec((_ROWBLK, _C), lambda i: (i, 0))
    full_spec = pl.BlockSpec((_C, _C), lambda i: (0, 0))
    vec_spec = pl.BlockSpec((1, _C), lambda i: (0, 0))
    b2 = b_conv.reshape(1, _C)
    s, q = pl.pallas_call(
        _proj_stats_body,
        grid=grid,
        in_specs=[row_spec, full_spec, vec_spec],
        out_specs=[vec_spec, vec_spec],
        out_shape=[jax.ShapeDtypeStruct((1, _C), jnp.float32)] * 2,
    )(agg, w_conv, b2)
    return pl.pallas_call(
        _proj_norm_body,
        grid=grid,
        in_specs=[row_spec, full_spec, vec_spec, vec_spec, vec_spec,
                  vec_spec, vec_spec],
        out_specs=row_spec,
        out_shape=jax.ShapeDtypeStruct((_P, _C), jnp.float32),
    )(agg, w_conv, b2, s, q, gamma.reshape(1, _C), beta.reshape(1, _C))


def kernel(feature, neigh_index, permatrix, W_mlp, W_conv, b_conv, gamma,
           beta):
    del permatrix
    b, c, n = feature.shape

    feat = jnp.transpose(feature, (0, 2, 1)).reshape(b * n, c)
    g = _attn_weights(feat, W_mlp)

    idx = jnp.pad(neigh_index, (0, _P_PAD * _K - neigh_index.shape[0]))
    idx_rows = idx.reshape(_P_PAD * _K // 128, 128)
    agg = _segsum(g, idx_rows)[:_P]

    out2d = _proj_bn(agg, W_conv, b_conv, gamma, beta)
    return jnp.transpose(out2d.reshape(b, n, -1), (0, 2, 1))


# speedup vs baseline: 1.1000x; 1.1000x over previous
import functools

import jax
import jax.numpy as jnp
from jax import lax
from jax.experimental import pallas as pl
from jax.experimental.pallas import tpu as pltpu
from jax.experimental.pallas import tpu_sc as plsc

_B, _C, _N, _K = 2, 128, 10000, 32
_P = _B * _N
_LANES = 16
_NW = 32
_EPC = 128
_PPC = _EPC // _K
_CPW = 160
_P_PAD = _NW * _CPW * _PPC
_FLUSH = 16
_ROWBLK = 1000


def _attn_weights_body(x_ref, w_ref, g_ref):
    x = x_ref[...]
    h = lax.dot_general(x, w_ref[...], (((1,), (1,)), ((), ())),
                        preferred_element_type=jnp.float32)
    h = h - jnp.max(h, axis=1, keepdims=True)
    e = jnp.exp(h)
    sm = e / jnp.sum(e, axis=1, keepdims=True)
    g_ref[...] = sm * x


def _attn_weights(feat, w_mlp):
    return pl.pallas_call(
        _attn_weights_body,
        grid=(_P // _ROWBLK,),
        in_specs=[
            pl.BlockSpec((_ROWBLK, _C), lambda i: (i, 0)),
            pl.BlockSpec((_C, _C), lambda i: (0, 0)),
        ],
        out_specs=pl.BlockSpec((_ROWBLK, _C), lambda i: (i, 0)),
        out_shape=jax.ShapeDtypeStruct((_P, _C), jnp.float32),
    )(feat, w_mlp)


_NBUF = 4
_CPW0 = 240
_CPW1 = 2 * _CPW - _CPW0


def _segsum_body(g_hbm, idx_hbm, out_hbm, idx_v, bufs, out_v, sems):
    cid = lax.axis_index("c")
    sid = lax.axis_index("s")
    chunk0 = sid * (2 * _CPW) + cid * _CPW0
    my_cpw = jnp.where(cid == 0, _CPW0, _CPW1)

    idx_row0 = pl.multiple_of(chunk0, 8)

    @pl.when(cid == 0)
    def _():
        pltpu.sync_copy(idx_hbm.at[pl.ds(idx_row0, _CPW0)], idx_v)

    @pl.when(cid != 0)
    def _():
        pltpu.sync_copy(idx_hbm.at[pl.ds(idx_row0, _CPW1)],
                        idx_v.at[pl.ds(0, _CPW1)])

    def start_gather(c, b):
        pltpu.async_copy(g_hbm.at[idx_v.at[c]], bufs[b], sems[b])

    def wait_gather(b):
        pltpu.make_async_copy(g_hbm.at[idx_v.at[0]], bufs[b], sems[b]).wait()

    def reduce_chunk(c, buf):
        orow = (c % _FLUSH) * _PPC
        for p in range(_PPC):
            accs = [buf[p * _K, pl.ds(l * _LANES, _LANES)]
                    for l in range(_C // _LANES)]
            for j in range(1, _K):
                accs = [accs[l] + buf[p * _K + j, pl.ds(l * _LANES, _LANES)]
                        for l in range(_C // _LANES)]
            for l in range(_C // _LANES):
                out_v[orow + p, pl.ds(l * _LANES, _LANES)] = accs[l]

    for b in range(_NBUF):
        start_gather(b, b)

    @pl.loop(0, _CPW0, step=_NBUF)
    def _group(c):
        for b in range(_NBUF):
            ch = c + b

            @pl.when(ch < my_cpw)
            def _():
                wait_gather(b)
                reduce_chunk(ch, bufs[b])

                @pl.when(ch + _NBUF < my_cpw)
                def _():
                    start_gather(ch + _NBUF, b)

            @pl.when((ch < my_cpw) & (ch % _FLUSH == _FLUSH - 1))
            def _():
                first_pt = pl.multiple_of(
                    (chunk0 + ch - (_FLUSH - 1)) * _PPC, 8)
                pltpu.sync_copy(out_v,
                                out_hbm.at[pl.ds(first_pt, _FLUSH * _PPC)])


def _segsum(g, idx_rows):
    mesh = plsc.VectorSubcoreMesh(core_axis_name="c", subcore_axis_name="s")

    def body(g_hbm, idx_hbm, out_hbm, idx_v, b0, b1, b2, b3, out_v,
             s0, s1, s2, s3):
        _segsum_body(g_hbm, idx_hbm, out_hbm, idx_v, (b0, b1, b2, b3),
                     out_v, (s0, s1, s2, s3))

    kern = pl.kernel(
        body,
        out_type=jax.ShapeDtypeStruct((_P_PAD, _C), jnp.float32),
        mesh=mesh,
        scratch_types=[
            pltpu.VMEM((_CPW0, _EPC), jnp.int32),
            pltpu.VMEM((_EPC, _C), jnp.float32),
            pltpu.VMEM((_EPC, _C), jnp.float32),
            pltpu.VMEM((_EPC, _C), jnp.float32),
            pltpu.VMEM((_EPC, _C), jnp.float32),
            pltpu.VMEM((_FLUSH * _PPC, _C), jnp.float32),
            pltpu.SemaphoreType.DMA,
            pltpu.SemaphoreType.DMA,
            pltpu.SemaphoreType.DMA,
            pltpu.SemaphoreType.DMA,
        ],
    )
    return kern(g, idx_rows)


def _proj_stats_body(a_ref, wc_ref, bc_ref, s_ref, q_ref):
    i = pl.program_id(0)
    a = a_ref[...].astype(jnp.float32)
    x = lax.dot_general(a, wc_ref[...], (((1,), (1,)), ((), ())),
                        preferred_element_type=jnp.float32) + bc_ref[...]

    @pl.when(i == 0)
    def _():
        s_ref[...] = jnp.zeros_like(s_ref)
        q_ref[...] = jnp.zeros_like(q_ref)

    s_ref[...] += jnp.sum(x, axis=0, keepdims=True)
    q_ref[...] += jnp.sum(x * x, axis=0, keepdims=True)


def _proj_norm_body(a_ref, wc_ref, bc_ref, s_ref, q_ref, gm_ref, bt_ref,
                    o_ref):
    a = a_ref[...].astype(jnp.float32)
    x = lax.dot_general(a, wc_ref[...], (((1,), (1,)), ((), ())),
                        preferred_element_type=jnp.float32) + bc_ref[...]
    inv_n = 1.0 / _P
    mean = s_ref[...] * inv_n
    var = q_ref[...] * inv_n - mean * mean
    scale = gm_ref[...] * lax.rsqrt(var + 1e-5)
    o_ref[...] = (x - mean) * scale + bt_ref[...]


def _proj_bn(agg, w_conv, b_conv, gamma, beta):
    grid = (_P // _ROWBLK,)
    row_spec = pl.BlockSpec((_ROWBLK, _C), lambda i: (i, 0))
    full_spec = pl.BlockSpec((_C, _C), lambda i: (0, 0))
    vec_spec = pl.BlockSpec((1, _C), lambda i: (0, 0))
    b2 = b_conv.reshape(1, _C)
    s, q = pl.pallas_call(
        _proj_stats_body,
        grid=grid,
        in_specs=[row_spec, full_spec, vec_spec],
        out_specs=[vec_spec, vec_spec],
        out_shape=[jax.ShapeDtypeStruct((1, _C), jnp.float32)] * 2,
    )(agg, w_conv, b2)
    return pl.pallas_call(
        _proj_norm_body,
        grid=grid,
        in_specs=[row_spec, full_spec, vec_spec, vec_spec, vec_spec,
                  vec_spec, vec_spec],
        out_specs=row_spec,
        out_shape=jax.ShapeDtypeStruct((_P, _C), jnp.float32),
    )(agg, w_conv, b2, s, q, gamma.reshape(1, _C), beta.reshape(1, _C))


def kernel(feature, neigh_index, permatrix, W_mlp, W_conv, b_conv, gamma,
           beta):
    del permatrix
    b, c, n = feature.shape

    feat = jnp.transpose(feature, (0, 2, 1)).reshape(b * n, c)
    g = _attn_weights(feat, W_mlp)

    idx = jnp.pad(neigh_index, (0, _P_PAD * _K - neigh_index.shape[0]))
    idx_rows = idx.reshape(_P_PAD * _K // 128, 128)
    agg = _segsum(g, idx_rows)[:_P]

    out2d = _proj_bn(agg, W_conv, b_conv, gamma, beta)
    return jnp.transpose(out2d.reshape(b, n, -1), (0, 2, 1))


# speedup vs baseline: 1.1099x; 1.0090x over previous
import functools

import jax
import jax.numpy as jnp
from jax import lax
from jax.experimental import pallas as pl
from jax.experimental.pallas import tpu as pltpu
from jax.experimental.pallas import tpu_sc as plsc

_B, _C, _N, _K = 2, 128, 10000, 32
_P = _B * _N
_LANES = 16
_NW = 32
_EPC = 128
_PPC = _EPC // _K
_CPW = 160
_P_PAD = _NW * _CPW * _PPC
_FLUSH = 16
_ROWBLK = 1000


def _attn_weights_body(x_ref, w_ref, g_ref):
    x = x_ref[...]
    h = lax.dot_general(x, w_ref[...], (((1,), (1,)), ((), ())),
                        preferred_element_type=jnp.float32)
    h = h - jnp.max(h, axis=1, keepdims=True)
    e = jnp.exp(h)
    sm = e / jnp.sum(e, axis=1, keepdims=True)
    g_ref[...] = sm * x


def _attn_weights(feat, w_mlp):
    return pl.pallas_call(
        _attn_weights_body,
        grid=(_P // _ROWBLK,),
        in_specs=[
            pl.BlockSpec((_ROWBLK, _C), lambda i: (i, 0)),
            pl.BlockSpec((_C, _C), lambda i: (0, 0)),
        ],
        out_specs=pl.BlockSpec((_ROWBLK, _C), lambda i: (i, 0)),
        out_shape=jax.ShapeDtypeStruct((_P, _C), jnp.float32),
    )(feat, w_mlp)


_NBUF = 4
_CPW0 = 240
_CPW1 = 2 * _CPW - _CPW0


def _segsum_body(g_hbm, idx_hbm, out_hbm, idx_v, bufs, out_v, sems):
    cid = lax.axis_index("c")
    sid = lax.axis_index("s")
    chunk0 = sid * (2 * _CPW) + cid * _CPW0
    my_cpw = jnp.where(cid == 0, _CPW0, _CPW1)

    idx_row0 = pl.multiple_of(chunk0, 8)

    @pl.when(cid == 0)
    def _():
        pltpu.sync_copy(idx_hbm.at[pl.ds(idx_row0, _CPW0)], idx_v)

    @pl.when(cid != 0)
    def _():
        pltpu.sync_copy(idx_hbm.at[pl.ds(idx_row0, _CPW1)],
                        idx_v.at[pl.ds(0, _CPW1)])

    def start_gather(c, b):
        pltpu.async_copy(g_hbm.at[idx_v.at[c]], bufs[b], sems[b])

    def wait_gather(b):
        pltpu.make_async_copy(g_hbm.at[idx_v.at[0]], bufs[b], sems[b]).wait()

    def reduce_chunk(c, buf):
        orow = (c % _FLUSH) * _PPC
        for p in range(_PPC):
            accs = [buf[p * _K, pl.ds(l * _LANES, _LANES)]
                    for l in range(_C // _LANES)]
            for j in range(1, _K):
                accs = [accs[l] + buf[p * _K + j, pl.ds(l * _LANES, _LANES)]
                        for l in range(_C // _LANES)]
            for l in range(_C // _LANES):
                out_v[orow + p, pl.ds(l * _LANES, _LANES)] = accs[l]

    for b in range(_NBUF):
        start_gather(b, b)

    @pl.loop(0, _CPW0, step=_NBUF)
    def _group(c):
        for b in range(_NBUF):
            ch = c + b

            @pl.when(ch < my_cpw)
            def _():
                wait_gather(b)
                reduce_chunk(ch, bufs[b])

                @pl.when(ch + _NBUF < my_cpw)
                def _():
                    start_gather(ch + _NBUF, b)

            @pl.when((ch < my_cpw) & (ch % _FLUSH == _FLUSH - 1))
            def _():
                first_pt = pl.multiple_of(
                    (chunk0 + ch - (_FLUSH - 1)) * _PPC, 8)
                pltpu.sync_copy(out_v,
                                out_hbm.at[pl.ds(first_pt, _FLUSH * _PPC)])


def _segsum(g, idx_rows):
    mesh = plsc.VectorSubcoreMesh(core_axis_name="c", subcore_axis_name="s")

    def body(g_hbm, idx_hbm, out_hbm, idx_v, b0, b1, b2, b3, out_v,
             s0, s1, s2, s3):
        _segsum_body(g_hbm, idx_hbm, out_hbm, idx_v, (b0, b1, b2, b3),
                     out_v, (s0, s1, s2, s3))

    kern = pl.kernel(
        body,
        out_type=jax.ShapeDtypeStruct((_P_PAD, _C), jnp.float32),
        mesh=mesh,
        scratch_types=[
            pltpu.VMEM((_CPW0, _EPC), jnp.int32),
            pltpu.VMEM((_EPC, _C), jnp.float32),
            pltpu.VMEM((_EPC, _C), jnp.float32),
            pltpu.VMEM((_EPC, _C), jnp.float32),
            pltpu.VMEM((_EPC, _C), jnp.float32),
            pltpu.VMEM((_FLUSH * _PPC, _C), jnp.float32),
            pltpu.SemaphoreType.DMA,
            pltpu.SemaphoreType.DMA,
            pltpu.SemaphoreType.DMA,
            pltpu.SemaphoreType.DMA,
        ],
    )
    return kern(g, idx_rows)


def _proj_stats_body(a_ref, wc_ref, bc_ref, s_ref, q_ref):
    i = pl.program_id(0)
    a = a_ref[...].astype(jnp.float32)
    x = lax.dot_general(a, wc_ref[...], (((1,), (1,)), ((), ())),
                        preferred_element_type=jnp.float32) + bc_ref[...]

    @pl.when(i == 0)
    def _():
        s_ref[...] = jnp.zeros_like(s_ref)
        q_ref[...] = jnp.zeros_like(q_ref)

    s_ref[...] += jnp.sum(x, axis=0, keepdims=True)
    q_ref[...] += jnp.sum(x * x, axis=0, keepdims=True)


def _proj_norm_body(a_ref, wc_ref, bc_ref, s_ref, q_ref, gm_ref, bt_ref,
                    o_ref):
    a = a_ref[...].astype(jnp.float32)
    x = lax.dot_general(a, wc_ref[...], (((1,), (1,)), ((), ())),
                        preferred_element_type=jnp.float32) + bc_ref[...]
    inv_n = 1.0 / _P
    mean = s_ref[...] * inv_n
    var = q_ref[...] * inv_n - mean * mean
    scale = gm_ref[...] * lax.rsqrt(var + 1e-5)
    o_ref[...] = (x - mean) * scale + bt_ref[...]


def _proj_bn(agg, w_conv, b_conv, gamma, beta):
    grid = (_P // _ROWBLK,)
    row_spec = pl.BlockSpec((_ROWBLK, _C), lambda i: (i, 0))
    full_spec = pl.BlockSpec((_C, _C), lambda i: (0, 0))
    vec_spec = pl.BlockSpec((1, _C), lambda i: (0, 0))
    b2 = b_conv.reshape(1, _C)
    s, q = pl.pallas_call(
        _proj_stats_body,
        grid=grid,
        in_specs=[row_spec, full_spec, vec_spec],
        out_specs=[vec_spec, vec_spec],
        out_shape=[jax.ShapeDtypeStruct((1, _C), jnp.float32)] * 2,
    )(agg, w_conv, b2)
    return pl.pallas_call(
        _proj_norm_body,
        grid=grid,
        in_specs=[row_spec, full_spec, vec_spec, vec_spec, vec_spec,
                  vec_spec, vec_spec],
        out_specs=row_spec,
        out_shape=jax.ShapeDtypeStruct((_P, _C), jnp.float32),
    )(agg, w_conv, b2, s, q, gamma.reshape(1, _C), beta.reshape(1, _C))


def kernel(feature, neigh_index, permatrix, W_mlp, W_conv, b_conv, gamma,
           beta):
    del permatrix
    b, c, n = feature.shape

    feat = jnp.transpose(feature, (0, 2, 1)).reshape(b * n, c)
    g = _attn_weights(feat, W_mlp)

    idx = jnp.pad(neigh_index, (0, _P_PAD * _K - neigh_index.shape[0]))
    idx_rows = idx.reshape(_P_PAD * _K // 128, 128)
    agg_pad = _segsum(g, idx_rows)

    out2d = _proj_bn(agg_pad, W_conv, b_conv, gamma, beta)
    return jnp.transpose(out2d.reshape(b, n, -1), (0, 2, 1))
